# baseline (speedup 1.0000x reference)
import functools

import jax
import jax.numpy as jnp
from jax.experimental import pallas as pl
from jax.experimental.pallas import tpu as pltpu

N = 4096
D = 32
KNN = 5

_HI = jax.lax.Precision.HIGHEST


def _d2_block(z_rows, z_cols):
    a2 = jnp.sum(z_rows * z_rows, axis=1, keepdims=True)
    b2 = jnp.sum(z_cols * z_cols, axis=1, keepdims=True)
    dot = jax.lax.dot_general(
        z_rows.astype(jnp.bfloat16), z_cols.astype(jnp.bfloat16),
        (((1,), (1,)), ((), ())),
        preferred_element_type=jnp.float32)
    d2 = a2 + b2.T - 2.0 * dot
    return jnp.maximum(d2, 0.0)


def _safe_sqrt(d2):
    return jnp.where(d2 > 0, jnp.sqrt(jnp.where(d2 > 0, d2, 1.0)), 0.0)


def _dist_block(z_rows, z_cols):
    return _safe_sqrt(_d2_block(z_rows, z_cols))


def _k1_kernel(z_ref, zi_ref, kth_ref):
    d2 = _d2_block(z_ref[...], zi_ref[...])
    remaining = d2
    count = jnp.zeros((1, d2.shape[1]), jnp.float32)
    kth = jnp.zeros((1, d2.shape[1]), jnp.float32)
    done = count >= KNN
    for _ in range(KNN):
        m = jnp.min(remaining, axis=0, keepdims=True)
        c = jnp.sum((remaining == m).astype(jnp.float32), axis=0, keepdims=True)
        newcount = count + c
        hit = jnp.logical_and(jnp.logical_not(done), newcount >= KNN)
        kth = jnp.where(hit, m, kth)
        done = jnp.logical_or(done, newcount >= KNN)
        remaining = jnp.where(remaining == m, jnp.inf, remaining)
        count = newcount
    kth_ref[...] = _safe_sqrt(kth).reshape(1, 1, d2.shape[1])


def _run_k1(z, cb=512):
    nblk = N // cb
    return pl.pallas_call(
        _k1_kernel,
        grid=(nblk,),
        in_specs=[
            pl.BlockSpec((N, D), lambda i: (0, 0)),
            pl.BlockSpec((cb, D), lambda i: (i, 0)),
        ],
        out_specs=pl.BlockSpec((1, 1, cb), lambda i: (i, 0, 0)),
        out_shape=jax.ShapeDtypeStruct((nblk, 1, cb), jnp.float32),
        compiler_params=pltpu.CompilerParams(
            dimension_semantics=("parallel",)),
    )(z, z)


def _k2_kernel(zi_ref, zj_ref, kthr_ref, kthc_ref, m_ref, r_ref, s_ref, rs_ref):
    j = pl.program_id(1)
    dist = _dist_block(zi_ref[...], zj_ref[...])
    kth_row = kthr_ref[...]
    kth_col = kthc_ref[...]
    mask = (dist <= kth_row).astype(jnp.float32)
    maskT = (dist <= kth_col).astype(jnp.float32)
    mask8 = mask.astype(jnp.float8_e4m3fn)
    rmat = mask * maskT
    m_ref[...] = mask8
    rmat8 = rmat.astype(jnp.float8_e4m3fn)
    r_ref[...] = rmat.astype(jnp.bfloat16)
    ones = jnp.full((mask8.shape[1], 128), 1.0, jnp.float8_e4m3fn)
    s_part = jax.lax.dot_general(mask8, ones, (((1,), (0,)), ((), ())),
                                 preferred_element_type=jnp.float32)
    r_part = jax.lax.dot_general(rmat8, ones, (((1,), (0,)), ((), ())),
                                 preferred_element_type=jnp.float32)

    @pl.when(j == 0)
    def _init():
        s_ref[...] = s_part
        rs_ref[...] = r_part

    @pl.when(j != 0)
    def _acc():
        s_ref[...] += s_part
        rs_ref[...] += r_part


def _run_k2(z, kth_r, kth_c, bm=1024, bn=2048):
    gi, gj = N // bm, N // bn
    return pl.pallas_call(
        _k2_kernel,
        grid=(gi, gj),
        in_specs=[
            pl.BlockSpec((bm, D), lambda i, j: (i, 0)),
            pl.BlockSpec((bn, D), lambda i, j: (j, 0)),
            pl.BlockSpec((1, bn), lambda i, j: (0, j)),
            pl.BlockSpec((bm, 1), lambda i, j: (i, 0)),
        ],
        out_specs=[
            pl.BlockSpec((bm, bn), lambda i, j: (i, j)),
            pl.BlockSpec((bm, bn), lambda i, j: (i, j)),
            pl.BlockSpec((bm, 128), lambda i, j: (i, 0)),
            pl.BlockSpec((bm, 128), lambda i, j: (i, 0)),
        ],
        out_shape=[
            jax.ShapeDtypeStruct((N, N), jnp.float8_e4m3fn),
            jax.ShapeDtypeStruct((N, N), jnp.bfloat16),
            jax.ShapeDtypeStruct((N, 128), jnp.float32),
            jax.ShapeDtypeStruct((N, 128), jnp.float32),
        ],
        compiler_params=pltpu.CompilerParams(
            dimension_semantics=("parallel", "arbitrary")),
    )(z, z, kth_r, kth_c)


def _k3_kernel(mi_ref, mj_ref, s_ref, sim_ref):
    p = jax.lax.dot_general(
        mi_ref[...], mj_ref[...], (((1,), (1,)), ((), ())),
        preferred_element_type=jnp.float32)
    sim_ref[...] = (p / s_ref[:, :1]).astype(jnp.bfloat16)


def _run_k3(m, s, bm=1024, bn=1024):
    gi, gj = N // bm, N // bn
    return pl.pallas_call(
        _k3_kernel,
        grid=(gi, gj),
        in_specs=[
            pl.BlockSpec((bm, N), lambda i, j: (i, 0)),
            pl.BlockSpec((bn, N), lambda i, j: (j, 0)),
            pl.BlockSpec((bm, 128), lambda i, j: (i, 0)),
        ],
        out_specs=pl.BlockSpec((bm, bn), lambda i, j: (i, j)),
        out_shape=jax.ShapeDtypeStruct((N, N), jnp.bfloat16),
        compiler_params=pltpu.CompilerParams(
            dimension_semantics=("parallel", "parallel")),
    )(m, m, s)


def _k4_kernel(sim_ref, r_ref, rs_ref, out_ref):
    acc = jax.lax.dot_general(sim_ref[...], r_ref[...],
                              (((1,), (0,)), ((), ())),
                              preferred_element_type=jnp.float32)
    out_ref[...] = acc / rs_ref[:, :1]


def _run_k4(sim, r, rs, bm=1024, bn=1024):
    gi, gj = N // bm, N // bn
    return pl.pallas_call(
        _k4_kernel,
        grid=(gi, gj),
        in_specs=[
            pl.BlockSpec((bm, N), lambda i, j: (i, 0)),
            pl.BlockSpec((N, bn), lambda i, j: (0, j)),
            pl.BlockSpec((bm, 128), lambda i, j: (i, 0)),
        ],
        out_specs=pl.BlockSpec((bm, bn), lambda i, j: (i, j)),
        out_shape=jax.ShapeDtypeStruct((N, N), jnp.float32),
        compiler_params=pltpu.CompilerParams(
            dimension_semantics=("parallel", "parallel")),
    )(sim, r, rs)


def _k5_kernel(a_ref, b_ref, out_ref):
    out_ref[...] = 0.5 * (a_ref[...] + b_ref[...].T)


def _run_k5(sim2, b=1024):
    g = N // b
    return pl.pallas_call(
        _k5_kernel,
        grid=(g, g),
        in_specs=[
            pl.BlockSpec((b, b), lambda i, j: (i, j)),
            pl.BlockSpec((b, b), lambda i, j: (j, i)),
        ],
        out_specs=pl.BlockSpec((b, b), lambda i, j: (i, j)),
        out_shape=jax.ShapeDtypeStruct((N, N), jnp.float32),
        compiler_params=pltpu.CompilerParams(
            dimension_semantics=("parallel", "parallel")),
    )(sim2, sim2)


@jax.jit
def kernel(z):
    kth = _run_k1(z)
    kth_flat = kth.reshape(N)
    kth_r = kth_flat.reshape(1, N)
    kth_c = kth_flat.reshape(N, 1)
    m, r, s, rs = _run_k2(z, kth_r, kth_c)
    sim = _run_k3(m, s)
    sim2 = _run_k4(sim, r, rs)
    return _run_k5(sim2)


# speedup vs baseline: 1.1651x; 1.1651x over previous
import functools

import jax
import jax.numpy as jnp
from jax.experimental import pallas as pl
from jax.experimental.pallas import tpu as pltpu

N = 4096
D = 32
KNN = 5

_HI = jax.lax.Precision.HIGHEST


def _d2_block(z_rows, z_cols):
    a2 = jnp.sum(z_rows * z_rows, axis=1, keepdims=True)
    b2 = jnp.sum(z_cols * z_cols, axis=1, keepdims=True)
    dot = jax.lax.dot_general(
        z_rows.astype(jnp.bfloat16), z_cols.astype(jnp.bfloat16),
        (((1,), (1,)), ((), ())),
        preferred_element_type=jnp.float32)
    d2 = a2 + b2.T - 2.0 * dot
    return jnp.maximum(d2, 0.0)


def _safe_sqrt(d2):
    return jnp.where(d2 > 0, jnp.sqrt(jnp.where(d2 > 0, d2, 1.0)), 0.0)


def _dist_block(z_rows, z_cols):
    return _safe_sqrt(_d2_block(z_rows, z_cols))


def _k1_kernel(z_ref, zi_ref, kth_ref):
    d2 = _d2_block(z_ref[...], zi_ref[...])
    remaining = d2
    count = jnp.zeros((1, d2.shape[1]), jnp.float32)
    kth = jnp.zeros((1, d2.shape[1]), jnp.float32)
    done = count >= KNN
    for _ in range(KNN):
        m = jnp.min(remaining, axis=0, keepdims=True)
        c = jnp.sum((remaining == m).astype(jnp.float32), axis=0, keepdims=True)
        newcount = count + c
        hit = jnp.logical_and(jnp.logical_not(done), newcount >= KNN)
        kth = jnp.where(hit, m, kth)
        done = jnp.logical_or(done, newcount >= KNN)
        remaining = jnp.where(remaining == m, jnp.inf, remaining)
        count = newcount
    kth_ref[...] = _safe_sqrt(kth).reshape(1, 1, d2.shape[1])


def _run_k1(z, cb=512):
    nblk = N // cb
    return pl.pallas_call(
        _k1_kernel,
        grid=(nblk,),
        in_specs=[
            pl.BlockSpec((N, D), lambda i: (0, 0)),
            pl.BlockSpec((cb, D), lambda i: (i, 0)),
        ],
        out_specs=pl.BlockSpec((1, 1, cb), lambda i: (i, 0, 0)),
        out_shape=jax.ShapeDtypeStruct((nblk, 1, cb), jnp.float32),
        compiler_params=pltpu.CompilerParams(
            dimension_semantics=("parallel",)),
    )(z, z)


def _k2_kernel(zi_ref, zj_ref, kthr_ref, kthc_ref, m_ref, r_ref, s_ref, rs_ref):
    j = pl.program_id(1)
    dist = _dist_block(zi_ref[...], zj_ref[...])
    kth_row = kthr_ref[...]
    kth_col = kthc_ref[...]
    mask = (dist <= kth_row).astype(jnp.float32)
    maskT = (dist <= kth_col).astype(jnp.float32)
    mask8 = mask.astype(jnp.float8_e4m3fn)
    rmat = mask * maskT
    m_ref[...] = mask8
    rmat8 = rmat.astype(jnp.float8_e4m3fn)
    r_ref[...] = rmat8
    ones = jnp.full((mask8.shape[1], 128), 1.0, jnp.float8_e4m3fn)
    s_part = jax.lax.dot_general(mask8, ones, (((1,), (0,)), ((), ())),
                                 preferred_element_type=jnp.float32)
    r_part = jax.lax.dot_general(rmat8, ones, (((1,), (0,)), ((), ())),
                                 preferred_element_type=jnp.float32)

    @pl.when(j == 0)
    def _init():
        s_ref[...] = s_part
        rs_ref[...] = r_part

    @pl.when(j != 0)
    def _acc():
        s_ref[...] += s_part
        rs_ref[...] += r_part


def _run_k2(z, kth_r, kth_c, bm=1024, bn=2048):
    gi, gj = N // bm, N // bn
    return pl.pallas_call(
        _k2_kernel,
        grid=(gi, gj),
        in_specs=[
            pl.BlockSpec((bm, D), lambda i, j: (i, 0)),
            pl.BlockSpec((bn, D), lambda i, j: (j, 0)),
            pl.BlockSpec((1, bn), lambda i, j: (0, j)),
            pl.BlockSpec((bm, 1), lambda i, j: (i, 0)),
        ],
        out_specs=[
            pl.BlockSpec((bm, bn), lambda i, j: (i, j)),
            pl.BlockSpec((bm, bn), lambda i, j: (i, j)),
            pl.BlockSpec((bm, 128), lambda i, j: (i, 0)),
            pl.BlockSpec((bm, 128), lambda i, j: (i, 0)),
        ],
        out_shape=[
            jax.ShapeDtypeStruct((N, N), jnp.float8_e4m3fn),
            jax.ShapeDtypeStruct((N, N), jnp.float8_e4m3fn),
            jax.ShapeDtypeStruct((N, 128), jnp.float32),
            jax.ShapeDtypeStruct((N, 128), jnp.float32),
        ],
        compiler_params=pltpu.CompilerParams(
            dimension_semantics=("parallel", "arbitrary")),
    )(z, z, kth_r, kth_c)


def _k3_kernel(ri_ref, mj_ref, h_ref):
    h = jax.lax.dot_general(
        ri_ref[...], mj_ref[...], (((1,), (0,)), ((), ())),
        preferred_element_type=jnp.float32)
    h_ref[...] = h.astype(jnp.float8_e4m3fn)


def _run_k3(r, m, bm=1024, bn=1024):
    gi, gj = N // bm, N // bn
    return pl.pallas_call(
        _k3_kernel,
        grid=(gi, gj),
        in_specs=[
            pl.BlockSpec((bm, N), lambda i, j: (i, 0)),
            pl.BlockSpec((N, bn), lambda i, j: (0, j)),
        ],
        out_specs=pl.BlockSpec((bm, bn), lambda i, j: (i, j)),
        out_shape=jax.ShapeDtypeStruct((N, N), jnp.float8_e4m3fn),
        compiler_params=pltpu.CompilerParams(
            dimension_semantics=("parallel", "parallel")),
    )(r, m)


def _k4_kernel(mi_ref, hj_ref, w_ref):
    w_ref[...] = jax.lax.dot_general(
        mi_ref[...], hj_ref[...], (((1,), (1,)), ((), ())),
        preferred_element_type=jnp.float32)


def _run_k4(m, h, bm=1024, bn=1024):
    gi, gj = N // bm, N // bn
    return pl.pallas_call(
        _k4_kernel,
        grid=(gi, gj),
        in_specs=[
            pl.BlockSpec((bm, N), lambda i, j: (i, 0)),
            pl.BlockSpec((bn, N), lambda i, j: (j, 0)),
        ],
        out_specs=pl.BlockSpec((bm, bn), lambda i, j: (i, j)),
        out_shape=jax.ShapeDtypeStruct((N, N), jnp.float32),
        compiler_params=pltpu.CompilerParams(
            dimension_semantics=("parallel", "parallel")),
    )(m, h)


def _k5_kernel(a_ref, b_ref, si_ref, ri_ref, sj_ref, rj_ref, out_ref):
    u_i = 1.0 / (si_ref[:, :1] * ri_ref[:, :1])
    u_j = 1.0 / (sj_ref[:, :1] * rj_ref[:, :1])
    out_ref[...] = 0.5 * (a_ref[...] * u_i + (b_ref[...] * u_j).T)


def _run_k5(w, s, rs, b=1024):
    g = N // b
    return pl.pallas_call(
        _k5_kernel,
        grid=(g, g),
        in_specs=[
            pl.BlockSpec((b, b), lambda i, j: (i, j)),
            pl.BlockSpec((b, b), lambda i, j: (j, i)),
            pl.BlockSpec((b, 128), lambda i, j: (i, 0)),
            pl.BlockSpec((b, 128), lambda i, j: (i, 0)),
            pl.BlockSpec((b, 128), lambda i, j: (j, 0)),
            pl.BlockSpec((b, 128), lambda i, j: (j, 0)),
        ],
        out_specs=pl.BlockSpec((b, b), lambda i, j: (i, j)),
        out_shape=jax.ShapeDtypeStruct((N, N), jnp.float32),
        compiler_params=pltpu.CompilerParams(
            dimension_semantics=("parallel", "parallel")),
    )(w, w, s, rs, s, rs)


@jax.jit
def kernel(z):
    kth = _run_k1(z)
    kth_flat = kth.reshape(N)
    kth_r = kth_flat.reshape(1, N)
    kth_c = kth_flat.reshape(N, 1)
    m, r, s, rs = _run_k2(z, kth_r, kth_c)
    h = _run_k3(r, m)
    w = _run_k4(m, h)
    return _run_k5(w, s, rs)


# speedup vs baseline: 1.2546x; 1.0768x over previous
import functools

import jax
import jax.numpy as jnp
from jax.experimental import pallas as pl
from jax.experimental.pallas import tpu as pltpu

N = 4096
D = 32
KNN = 5

_HI = jax.lax.Precision.HIGHEST


def _d2_block(z_rows, z_cols):
    a2 = jnp.sum(z_rows * z_rows, axis=1, keepdims=True)
    b2 = jnp.sum(z_cols * z_cols, axis=1, keepdims=True)
    dot = jax.lax.dot_general(
        z_rows.astype(jnp.bfloat16), z_cols.astype(jnp.bfloat16),
        (((1,), (1,)), ((), ())),
        preferred_element_type=jnp.float32)
    return a2 + b2.T - 2.0 * dot


def _safe_sqrt(d2):
    return jnp.where(d2 > 0, jnp.sqrt(jnp.where(d2 > 0, d2, 1.0)), 0.0)


def _dist_block(z_rows, z_cols):
    return _safe_sqrt(_d2_block(z_rows, z_cols))


def _k1_kernel(z_ref, zi_ref, kth_ref):
    d2 = _d2_block(z_ref[...], zi_ref[...])
    remaining = d2
    count = jnp.zeros((1, d2.shape[1]), jnp.float32)
    kth = jnp.zeros((1, d2.shape[1]), jnp.float32)
    done = count >= KNN
    ones_row = jnp.full((1, d2.shape[0]), 1.0, jnp.bfloat16)
    for _ in range(KNN):
        m = jnp.min(remaining, axis=0, keepdims=True)
        c = jax.lax.dot_general(
            ones_row, (remaining == m).astype(jnp.bfloat16),
            (((1,), (0,)), ((), ())), preferred_element_type=jnp.float32)
        newcount = count + c
        hit = jnp.logical_and(jnp.logical_not(done), newcount >= KNN)
        kth = jnp.where(hit, m, kth)
        done = jnp.logical_or(done, newcount >= KNN)
        remaining = jnp.where(remaining == m, jnp.inf, remaining)
        count = newcount
    kth_ref[...] = _safe_sqrt(kth).reshape(1, 1, d2.shape[1])


def _run_k1(z, cb=512):
    nblk = N // cb
    return pl.pallas_call(
        _k1_kernel,
        grid=(nblk,),
        in_specs=[
            pl.BlockSpec((N, D), lambda i: (0, 0)),
            pl.BlockSpec((cb, D), lambda i: (i, 0)),
        ],
        out_specs=pl.BlockSpec((1, 1, cb), lambda i: (i, 0, 0)),
        out_shape=jax.ShapeDtypeStruct((nblk, 1, cb), jnp.float32),
        compiler_params=pltpu.CompilerParams(
            dimension_semantics=("parallel",)),
    )(z, z)


def _k2_kernel(zi_ref, zj_ref, kthr_ref, kthc_ref, m_ref, r_ref, s_ref, rs_ref):
    j = pl.program_id(1)
    dist = _dist_block(zi_ref[...], zj_ref[...])
    kth_row = kthr_ref[...]
    kth_col = kthc_ref[...]
    mask = (dist <= kth_row).astype(jnp.float32)
    maskT = (dist <= kth_col).astype(jnp.float32)
    mask8 = mask.astype(jnp.float8_e4m3fn)
    rmat = mask * maskT
    m_ref[...] = mask8
    rmat8 = rmat.astype(jnp.float8_e4m3fn)
    r_ref[...] = rmat8
    ones = jnp.full((mask8.shape[1], 128), 1.0, jnp.float8_e4m3fn)
    s_part = jax.lax.dot_general(mask8, ones, (((1,), (0,)), ((), ())),
                                 preferred_element_type=jnp.float32)
    r_part = jax.lax.dot_general(rmat8, ones, (((1,), (0,)), ((), ())),
                                 preferred_element_type=jnp.float32)

    @pl.when(j == 0)
    def _init():
        s_ref[...] = s_part
        rs_ref[...] = r_part

    @pl.when(j != 0)
    def _acc():
        s_ref[...] += s_part
        rs_ref[...] += r_part


def _run_k2(z, kth_r, kth_c, bm=1024, bn=2048):
    gi, gj = N // bm, N // bn
    return pl.pallas_call(
        _k2_kernel,
        grid=(gi, gj),
        in_specs=[
            pl.BlockSpec((bm, D), lambda i, j: (i, 0)),
            pl.BlockSpec((bn, D), lambda i, j: (j, 0)),
            pl.BlockSpec((1, bn), lambda i, j: (0, j)),
            pl.BlockSpec((bm, 1), lambda i, j: (i, 0)),
        ],
        out_specs=[
            pl.BlockSpec((bm, bn), lambda i, j: (i, j)),
            pl.BlockSpec((bm, bn), lambda i, j: (i, j)),
            pl.BlockSpec((bm, 128), lambda i, j: (i, 0)),
            pl.BlockSpec((bm, 128), lambda i, j: (i, 0)),
        ],
        out_shape=[
            jax.ShapeDtypeStruct((N, N), jnp.float8_e4m3fn),
            jax.ShapeDtypeStruct((N, N), jnp.float8_e4m3fn),
            jax.ShapeDtypeStruct((N, 128), jnp.float32),
            jax.ShapeDtypeStruct((N, 128), jnp.float32),
        ],
        compiler_params=pltpu.CompilerParams(
            dimension_semantics=("parallel", "arbitrary")),
    )(z, z, kth_r, kth_c)


def _k3_kernel(ri_ref, mj_ref, h_ref):
    h = jax.lax.dot_general(
        ri_ref[...], mj_ref[...], (((1,), (0,)), ((), ())),
        preferred_element_type=jnp.float32)
    h_ref[...] = h.astype(jnp.float8_e4m3fn)


def _run_k3(r, m, bm=1024, bn=1024):
    gi, gj = N // bm, N // bn
    return pl.pallas_call(
        _k3_kernel,
        grid=(gi, gj),
        in_specs=[
            pl.BlockSpec((bm, N), lambda i, j: (i, 0)),
            pl.BlockSpec((N, bn), lambda i, j: (0, j)),
        ],
        out_specs=pl.BlockSpec((bm, bn), lambda i, j: (i, j)),
        out_shape=jax.ShapeDtypeStruct((N, N), jnp.float8_e4m3fn),
        compiler_params=pltpu.CompilerParams(
            dimension_semantics=("parallel", "parallel")),
    )(r, m)


def _k4_kernel(mi_ref, hj_ref, w_ref):
    w_ref[...] = jax.lax.dot_general(
        mi_ref[...], hj_ref[...], (((1,), (1,)), ((), ())),
        preferred_element_type=jnp.float32)


def _run_k4(m, h, bm=1024, bn=1024):
    gi, gj = N // bm, N // bn
    return pl.pallas_call(
        _k4_kernel,
        grid=(gi, gj),
        in_specs=[
            pl.BlockSpec((bm, N), lambda i, j: (i, 0)),
            pl.BlockSpec((bn, N), lambda i, j: (j, 0)),
        ],
        out_specs=pl.BlockSpec((bm, bn), lambda i, j: (i, j)),
        out_shape=jax.ShapeDtypeStruct((N, N), jnp.float32),
        compiler_params=pltpu.CompilerParams(
            dimension_semantics=("parallel", "parallel")),
    )(m, h)


def _k5_kernel(a_ref, b_ref, si_ref, ri_ref, sj_ref, rj_ref, out_ref):
    u_i = 1.0 / (si_ref[:, :1] * ri_ref[:, :1])
    u_j = 1.0 / (sj_ref[:, :1] * rj_ref[:, :1])
    out_ref[...] = 0.5 * (a_ref[...] * u_i + (b_ref[...] * u_j).T)


def _run_k5(w, s, rs, b=1024):
    g = N // b
    return pl.pallas_call(
        _k5_kernel,
        grid=(g, g),
        in_specs=[
            pl.BlockSpec((b, b), lambda i, j: (i, j)),
            pl.BlockSpec((b, b), lambda i, j: (j, i)),
            pl.BlockSpec((b, 128), lambda i, j: (i, 0)),
            pl.BlockSpec((b, 128), lambda i, j: (i, 0)),
            pl.BlockSpec((b, 128), lambda i, j: (j, 0)),
            pl.BlockSpec((b, 128), lambda i, j: (j, 0)),
        ],
        out_specs=pl.BlockSpec((b, b), lambda i, j: (i, j)),
        out_shape=jax.ShapeDtypeStruct((N, N), jnp.float32),
        compiler_params=pltpu.CompilerParams(
            dimension_semantics=("parallel", "parallel")),
    )(w, w, s, rs, s, rs)


@jax.jit
def kernel(z):
    kth = _run_k1(z)
    kth_flat = kth.reshape(N)
    kth_r = kth_flat.reshape(1, N)
    kth_c = kth_flat.reshape(N, 1)
    m, r, s, rs = _run_k2(z, kth_r, kth_c)
    h = _run_k3(r, m)
    w = _run_k4(m, h)
    return _run_k5(w, s, rs)


# speedup vs baseline: 1.3656x; 1.0885x over previous
import functools

import jax
import jax.numpy as jnp
from jax.experimental import pallas as pl
from jax.experimental.pallas import tpu as pltpu

N = 4096
D = 32
KNN = 5

_HI = jax.lax.Precision.HIGHEST


def _d2_block(z_rows, z_cols):
    a2 = jnp.sum(z_rows * z_rows, axis=1, keepdims=True)
    b2 = jnp.sum(z_cols * z_cols, axis=1, keepdims=True)
    dot = jax.lax.dot_general(
        z_rows.astype(jnp.bfloat16), z_cols.astype(jnp.bfloat16),
        (((1,), (1,)), ((), ())),
        preferred_element_type=jnp.float32)
    return a2 + b2.T - 2.0 * dot


def _safe_sqrt(d2):
    return jnp.where(d2 > 0, jnp.sqrt(jnp.where(d2 > 0, d2, 1.0)), 0.0)


def _k1_kernel(z_ref, zi_ref, kth_ref):
    d2 = _d2_block(z_ref[...], zi_ref[...])
    remaining = d2
    count = jnp.zeros((1, d2.shape[1]), jnp.float32)
    kth = jnp.zeros((1, d2.shape[1]), jnp.float32)
    done = count >= KNN
    ones_row = jnp.full((1, d2.shape[0]), 1.0, jnp.bfloat16)
    for _ in range(KNN):
        m = jnp.min(remaining, axis=0, keepdims=True)
        c = jax.lax.dot_general(
            ones_row, (remaining == m).astype(jnp.bfloat16),
            (((1,), (0,)), ((), ())), preferred_element_type=jnp.float32)
        newcount = count + c
        hit = jnp.logical_and(jnp.logical_not(done), newcount >= KNN)
        kth = jnp.where(hit, m, kth)
        done = jnp.logical_or(done, newcount >= KNN)
        remaining = jnp.where(remaining == m, jnp.inf, remaining)
        count = newcount
    t = _safe_sqrt(kth)
    u = t * t
    ubits = jax.lax.bitcast_convert_type(u, jnp.int32)
    mprime = jnp.full_like(u, -jnp.inf)
    for k in range(-5, 6):
        c = jax.lax.bitcast_convert_type(ubits + k, jnp.float32)
        mprime = jnp.maximum(mprime, jnp.where(_safe_sqrt(c) <= t, c, -jnp.inf))
    kth_ref[...] = mprime.reshape(1, 1, d2.shape[1])


def _run_k1(z, cb=512):
    nblk = N // cb
    return pl.pallas_call(
        _k1_kernel,
        grid=(nblk,),
        in_specs=[
            pl.BlockSpec((N, D), lambda i: (0, 0)),
            pl.BlockSpec((cb, D), lambda i: (i, 0)),
        ],
        out_specs=pl.BlockSpec((1, 1, cb), lambda i: (i, 0, 0)),
        out_shape=jax.ShapeDtypeStruct((nblk, 1, cb), jnp.float32),
        compiler_params=pltpu.CompilerParams(
            dimension_semantics=("parallel",)),
    )(z, z)


def _k2_kernel(zi_ref, zj_ref, kthr_ref, kthc_ref, m_ref, r_ref, s_ref, rs_ref):
    j = pl.program_id(1)
    d2 = _d2_block(zi_ref[...], zj_ref[...])
    thr_row = kthr_ref[...]
    thr_col = kthc_ref[...]
    in_row = d2 <= thr_row
    in_col = d2 <= thr_col
    mask = in_row.astype(jnp.float32)
    rmat = jnp.logical_and(in_row, in_col).astype(jnp.float32)
    mask8 = mask.astype(jnp.float8_e4m3fn)
    m_ref[...] = mask8
    rmat8 = rmat.astype(jnp.float8_e4m3fn)
    r_ref[...] = rmat8
    ones = jnp.full((mask8.shape[1], 128), 1.0, jnp.float8_e4m3fn)
    s_part = jax.lax.dot_general(mask8, ones, (((1,), (0,)), ((), ())),
                                 preferred_element_type=jnp.float32)
    r_part = jax.lax.dot_general(rmat8, ones, (((1,), (0,)), ((), ())),
                                 preferred_element_type=jnp.float32)

    @pl.when(j == 0)
    def _init():
        s_ref[...] = s_part
        rs_ref[...] = r_part

    @pl.when(j != 0)
    def _acc():
        s_ref[...] += s_part
        rs_ref[...] += r_part


def _run_k2(z, kth_r, kth_c, bm=1024, bn=2048):
    gi, gj = N // bm, N // bn
    return pl.pallas_call(
        _k2_kernel,
        grid=(gi, gj),
        in_specs=[
            pl.BlockSpec((bm, D), lambda i, j: (i, 0)),
            pl.BlockSpec((bn, D), lambda i, j: (j, 0)),
            pl.BlockSpec((1, bn), lambda i, j: (0, j)),
            pl.BlockSpec((bm, 1), lambda i, j: (i, 0)),
        ],
        out_specs=[
            pl.BlockSpec((bm, bn), lambda i, j: (i, j)),
            pl.BlockSpec((bm, bn), lambda i, j: (i, j)),
            pl.BlockSpec((bm, 128), lambda i, j: (i, 0)),
            pl.BlockSpec((bm, 128), lambda i, j: (i, 0)),
        ],
        out_shape=[
            jax.ShapeDtypeStruct((N, N), jnp.float8_e4m3fn),
            jax.ShapeDtypeStruct((N, N), jnp.float8_e4m3fn),
            jax.ShapeDtypeStruct((N, 128), jnp.float32),
            jax.ShapeDtypeStruct((N, 128), jnp.float32),
        ],
        compiler_params=pltpu.CompilerParams(
            dimension_semantics=("parallel", "arbitrary")),
    )(z, z, kth_r, kth_c)


def _k3_kernel(ri_ref, mj_ref, h_ref):
    h = jax.lax.dot_general(
        ri_ref[...], mj_ref[...], (((1,), (0,)), ((), ())),
        preferred_element_type=jnp.float32)
    h_ref[...] = h.astype(jnp.float8_e4m3fn)


def _run_k3(r, m, bm=1024, bn=1024):
    gi, gj = N // bm, N // bn
    return pl.pallas_call(
        _k3_kernel,
        grid=(gi, gj),
        in_specs=[
            pl.BlockSpec((bm, N), lambda i, j: (i, 0)),
            pl.BlockSpec((N, bn), lambda i, j: (0, j)),
        ],
        out_specs=pl.BlockSpec((bm, bn), lambda i, j: (i, j)),
        out_shape=jax.ShapeDtypeStruct((N, N), jnp.float8_e4m3fn),
        compiler_params=pltpu.CompilerParams(
            dimension_semantics=("parallel", "parallel")),
    )(r, m)


def _k4_kernel(mi_ref, hj_ref, w_ref):
    w_ref[...] = jax.lax.dot_general(
        mi_ref[...], hj_ref[...], (((1,), (1,)), ((), ())),
        preferred_element_type=jnp.float32)


def _run_k4(m, h, bm=1024, bn=1024):
    gi, gj = N // bm, N // bn
    return pl.pallas_call(
        _k4_kernel,
        grid=(gi, gj),
        in_specs=[
            pl.BlockSpec((bm, N), lambda i, j: (i, 0)),
            pl.BlockSpec((bn, N), lambda i, j: (j, 0)),
        ],
        out_specs=pl.BlockSpec((bm, bn), lambda i, j: (i, j)),
        out_shape=jax.ShapeDtypeStruct((N, N), jnp.float32),
        compiler_params=pltpu.CompilerParams(
            dimension_semantics=("parallel", "parallel")),
    )(m, h)


def _k5_kernel(a_ref, b_ref, si_ref, ri_ref, sj_ref, rj_ref, out_ref):
    u_i = 1.0 / (si_ref[:, :1] * ri_ref[:, :1])
    u_j = 1.0 / (sj_ref[:, :1] * rj_ref[:, :1])
    out_ref[...] = 0.5 * (a_ref[...] * u_i + (b_ref[...] * u_j).T)


def _run_k5(w, s, rs, b=1024):
    g = N // b
    return pl.pallas_call(
        _k5_kernel,
        grid=(g, g),
        in_specs=[
            pl.BlockSpec((b, b), lambda i, j: (i, j)),
            pl.BlockSpec((b, b), lambda i, j: (j, i)),
            pl.BlockSpec((b, 128), lambda i, j: (i, 0)),
            pl.BlockSpec((b, 128), lambda i, j: (i, 0)),
            pl.BlockSpec((b, 128), lambda i, j: (j, 0)),
            pl.BlockSpec((b, 128), lambda i, j: (j, 0)),
        ],
        out_specs=pl.BlockSpec((b, b), lambda i, j: (i, j)),
        out_shape=jax.ShapeDtypeStruct((N, N), jnp.float32),
        compiler_params=pltpu.CompilerParams(
            dimension_semantics=("parallel", "parallel")),
    )(w, w, s, rs, s, rs)


@jax.jit
def kernel(z):
    kth = _run_k1(z)
    kth_flat = kth.reshape(N)
    kth_r = kth_flat.reshape(1, N)
    kth_c = kth_flat.reshape(N, 1)
    m, r, s, rs = _run_k2(z, kth_r, kth_c)
    h = _run_k3(r, m)
    w = _run_k4(m, h)
    return _run_k5(w, s, rs)


# speedup vs baseline: 1.4411x; 1.0553x over previous
import functools

import jax
import jax.numpy as jnp
from jax.experimental import pallas as pl
from jax.experimental.pallas import tpu as pltpu

N = 4096
D = 32
KNN = 5

_HI = jax.lax.Precision.HIGHEST


def _d2_block(z_rows, z_cols):
    a2 = jnp.sum(z_rows * z_rows, axis=1, keepdims=True)
    b2 = jnp.sum(z_cols * z_cols, axis=1, keepdims=True)
    dot = jax.lax.dot_general(
        z_rows.astype(jnp.bfloat16), z_cols.astype(jnp.bfloat16),
        (((1,), (1,)), ((), ())),
        preferred_element_type=jnp.float32)
    return a2 + b2.T - 2.0 * dot


def _safe_sqrt(d2):
    return jnp.where(d2 > 0, jnp.sqrt(jnp.where(d2 > 0, d2, 1.0)), 0.0)


def _k1_kernel(z_ref, zi_ref, kth_ref):
    d2 = _d2_block(z_ref[...], zi_ref[...])
    remaining = d2
    count = jnp.zeros((1, d2.shape[1]), jnp.float32)
    kth = jnp.zeros((1, d2.shape[1]), jnp.float32)
    done = count >= KNN
    ones_row = jnp.full((1, d2.shape[0]), 1.0, jnp.bfloat16)
    for t in range(KNN):
        m = jnp.min(remaining, axis=0, keepdims=True)
        c = jax.lax.dot_general(
            ones_row, (remaining == m).astype(jnp.bfloat16),
            (((1,), (0,)), ((), ())), preferred_element_type=jnp.float32)
        newcount = count + c
        hit = jnp.logical_and(jnp.logical_not(done), newcount >= KNN)
        kth = jnp.where(hit, m, kth)
        done = jnp.logical_or(done, newcount >= KNN)
        if t + 1 < KNN:
            remaining = jnp.where(remaining == m, jnp.inf, remaining)
        count = newcount
    t = _safe_sqrt(kth)
    u = t * t
    ubits = jax.lax.bitcast_convert_type(u, jnp.int32)
    mprime = jnp.full_like(u, -jnp.inf)
    for k in range(-5, 6):
        c = jax.lax.bitcast_convert_type(ubits + k, jnp.float32)
        mprime = jnp.maximum(mprime, jnp.where(_safe_sqrt(c) <= t, c, -jnp.inf))
    kth_ref[...] = mprime.reshape(1, 1, d2.shape[1])


def _run_k1(z, cb=512):
    nblk = N // cb
    return pl.pallas_call(
        _k1_kernel,
        grid=(nblk,),
        in_specs=[
            pl.BlockSpec((N, D), lambda i: (0, 0)),
            pl.BlockSpec((cb, D), lambda i: (i, 0)),
        ],
        out_specs=pl.BlockSpec((1, 1, cb), lambda i: (i, 0, 0)),
        out_shape=jax.ShapeDtypeStruct((nblk, 1, cb), jnp.float32),
        compiler_params=pltpu.CompilerParams(
            dimension_semantics=("parallel",)),
    )(z, z)


def _k2_kernel(zi_ref, zj_ref, kthr_ref, kthc_ref, m_ref, r_ref, s_ref, rs_ref):
    j = pl.program_id(1)
    d2 = _d2_block(zi_ref[...], zj_ref[...])
    thr_row = kthr_ref[...]
    thr_col = kthc_ref[...]
    in_row = d2 <= thr_row
    in_col = d2 <= thr_col
    mask = in_row.astype(jnp.float32)
    rmat = jnp.logical_and(in_row, in_col).astype(jnp.float32)
    mask8 = mask.astype(jnp.float8_e4m3fn)
    m_ref[...] = mask8
    rmat8 = rmat.astype(jnp.float8_e4m3fn)
    r_ref[...] = rmat8
    ones = jnp.full((mask8.shape[1], 128), 1.0, jnp.float8_e4m3fn)
    s_part = jax.lax.dot_general(mask8, ones, (((1,), (0,)), ((), ())),
                                 preferred_element_type=jnp.float32)
    r_part = jax.lax.dot_general(rmat8, ones, (((1,), (0,)), ((), ())),
                                 preferred_element_type=jnp.float32)

    @pl.when(j == 0)
    def _init():
        s_ref[...] = s_part
        rs_ref[...] = r_part

    @pl.when(j != 0)
    def _acc():
        s_ref[...] += s_part
        rs_ref[...] += r_part


def _run_k2(z, kth_r, kth_c, bm=1024, bn=2048):
    gi, gj = N // bm, N // bn
    return pl.pallas_call(
        _k2_kernel,
        grid=(gi, gj),
        in_specs=[
            pl.BlockSpec((bm, D), lambda i, j: (i, 0)),
            pl.BlockSpec((bn, D), lambda i, j: (j, 0)),
            pl.BlockSpec((1, bn), lambda i, j: (0, j)),
            pl.BlockSpec((bm, 1), lambda i, j: (i, 0)),
        ],
        out_specs=[
            pl.BlockSpec((bm, bn), lambda i, j: (i, j)),
            pl.BlockSpec((bm, bn), lambda i, j: (i, j)),
            pl.BlockSpec((bm, 128), lambda i, j: (i, 0)),
            pl.BlockSpec((bm, 128), lambda i, j: (i, 0)),
        ],
        out_shape=[
            jax.ShapeDtypeStruct((N, N), jnp.float8_e4m3fn),
            jax.ShapeDtypeStruct((N, N), jnp.float8_e4m3fn),
            jax.ShapeDtypeStruct((N, 128), jnp.float32),
            jax.ShapeDtypeStruct((N, 128), jnp.float32),
        ],
        compiler_params=pltpu.CompilerParams(
            dimension_semantics=("parallel", "arbitrary")),
    )(z, z, kth_r, kth_c)


def _k3_kernel(ri_ref, mj_ref, h_ref):
    h = jax.lax.dot_general(
        ri_ref[...], mj_ref[...], (((1,), (0,)), ((), ())),
        preferred_element_type=jnp.float32)
    h_ref[...] = h.astype(jnp.float8_e4m3fn)


def _run_k3(r, m, bm=1024, bn=2048):
    gi, gj = N // bm, N // bn
    return pl.pallas_call(
        _k3_kernel,
        grid=(gi, gj),
        in_specs=[
            pl.BlockSpec((bm, N), lambda i, j: (i, 0)),
            pl.BlockSpec((N, bn), lambda i, j: (0, j)),
        ],
        out_specs=pl.BlockSpec((bm, bn), lambda i, j: (i, j)),
        out_shape=jax.ShapeDtypeStruct((N, N), jnp.float8_e4m3fn),
        compiler_params=pltpu.CompilerParams(
            dimension_semantics=("parallel", "parallel")),
    )(r, m)


def _k4_kernel(mi_ref, hj_ref, w_ref):
    w = jax.lax.dot_general(
        mi_ref[...], hj_ref[...], (((1,), (1,)), ((), ())),
        preferred_element_type=jnp.float32)
    w_ref[...] = w.astype(jnp.bfloat16)


def _run_k4(m, h, bm=1024, bn=2048):
    gi, gj = N // bm, N // bn
    return pl.pallas_call(
        _k4_kernel,
        grid=(gi, gj),
        in_specs=[
            pl.BlockSpec((bm, N), lambda i, j: (i, 0)),
            pl.BlockSpec((bn, N), lambda i, j: (j, 0)),
        ],
        out_specs=pl.BlockSpec((bm, bn), lambda i, j: (i, j)),
        out_shape=jax.ShapeDtypeStruct((N, N), jnp.bfloat16),
        compiler_params=pltpu.CompilerParams(
            dimension_semantics=("parallel", "parallel")),
    )(m, h)


def _k5_kernel(a_ref, b_ref, si_ref, ri_ref, sj_ref, rj_ref, out_ref):
    u_i = 1.0 / (si_ref[:, :1] * ri_ref[:, :1])
    u_j = 1.0 / (sj_ref[:, :1] * rj_ref[:, :1])
    a = a_ref[...].astype(jnp.float32)
    b = b_ref[...].astype(jnp.float32)
    out_ref[...] = 0.5 * (a * u_i + (b * u_j).T)


def _run_k5(w, s, rs, b=1024):
    g = N // b
    return pl.pallas_call(
        _k5_kernel,
        grid=(g, g),
        in_specs=[
            pl.BlockSpec((b, b), lambda i, j: (i, j)),
            pl.BlockSpec((b, b), lambda i, j: (j, i)),
            pl.BlockSpec((b, 128), lambda i, j: (i, 0)),
            pl.BlockSpec((b, 128), lambda i, j: (i, 0)),
            pl.BlockSpec((b, 128), lambda i, j: (j, 0)),
            pl.BlockSpec((b, 128), lambda i, j: (j, 0)),
        ],
        out_specs=pl.BlockSpec((b, b), lambda i, j: (i, j)),
        out_shape=jax.ShapeDtypeStruct((N, N), jnp.float32),
        compiler_params=pltpu.CompilerParams(
            dimension_semantics=("parallel", "parallel")),
    )(w, w, s, rs, s, rs)


@jax.jit
def kernel(z):
    kth = _run_k1(z)
    kth_flat = kth.reshape(N)
    kth_r = kth_flat.reshape(1, N)
    kth_c = kth_flat.reshape(N, 1)
    m, r, s, rs = _run_k2(z, kth_r, kth_c)
    h = _run_k3(r, m)
    w = _run_k4(m, h)
    return _run_k5(w, s, rs)


# speedup vs baseline: 1.4426x; 1.0011x over previous
import jax
import jax.numpy as jnp
from jax.experimental import pallas as pl
from jax.experimental.pallas import tpu as pltpu

N = 4096
D = 32
KNN = 5


def _d2_block(z_rows, z_cols):
    a2 = jnp.sum(z_rows * z_rows, axis=1, keepdims=True)
    b2 = jnp.sum(z_cols * z_cols, axis=1, keepdims=True)
    dot = jax.lax.dot_general(
        z_rows.astype(jnp.bfloat16), z_cols.astype(jnp.bfloat16),
        (((1,), (1,)), ((), ())),
        preferred_element_type=jnp.float32)
    return a2 + b2.T - 2.0 * dot


def _safe_sqrt(d2):
    return jnp.where(d2 > 0, jnp.sqrt(jnp.where(d2 > 0, d2, 1.0)), 0.0)


def _k1_kernel(z_ref, zi_ref, kth_ref):
    d2 = _d2_block(z_ref[...], zi_ref[...])
    remaining = d2
    count = jnp.zeros((1, d2.shape[1]), jnp.float32)
    kth = jnp.zeros((1, d2.shape[1]), jnp.float32)
    done = count >= KNN
    ones_row = jnp.full((1, d2.shape[0]), 1.0, jnp.bfloat16)
    for t in range(KNN):
        m = jnp.min(remaining, axis=0, keepdims=True)
        c = jax.lax.dot_general(
            ones_row, (remaining == m).astype(jnp.bfloat16),
            (((1,), (0,)), ((), ())), preferred_element_type=jnp.float32)
        newcount = count + c
        hit = jnp.logical_and(jnp.logical_not(done), newcount >= KNN)
        kth = jnp.where(hit, m, kth)
        done = jnp.logical_or(done, newcount >= KNN)
        if t + 1 < KNN:
            remaining = jnp.where(remaining == m, jnp.inf, remaining)
        count = newcount
    t = _safe_sqrt(kth)
    u = t * t
    ubits = jax.lax.bitcast_convert_type(u, jnp.int32)
    mprime = jnp.full_like(u, -jnp.inf)
    for k in range(-5, 6):
        c = jax.lax.bitcast_convert_type(ubits + k, jnp.float32)
        mprime = jnp.maximum(mprime, jnp.where(_safe_sqrt(c) <= t, c, -jnp.inf))
    kth_ref[...] = mprime.reshape(1, 1, d2.shape[1])


def _run_k1(z, cb=512):
    nblk = N // cb
    return pl.pallas_call(
        _k1_kernel,
        grid=(nblk,),
        in_specs=[
            pl.BlockSpec((N, D), lambda i: (0, 0)),
            pl.BlockSpec((cb, D), lambda i: (i, 0)),
        ],
        out_specs=pl.BlockSpec((1, 1, cb), lambda i: (i, 0, 0)),
        out_shape=jax.ShapeDtypeStruct((nblk, 1, cb), jnp.float32),
        compiler_params=pltpu.CompilerParams(
            dimension_semantics=("parallel",)),
    )(z, z)


def _k2_kernel(zi_ref, zj_ref, kthr_ref, kthc_ref, m_ref, r_ref, s_ref, rs_ref):
    j = pl.program_id(1)
    d2 = _d2_block(zi_ref[...], zj_ref[...])
    thr_row = kthr_ref[...]
    thr_col = kthc_ref[...]
    in_row = d2 <= thr_row
    in_col = d2 <= thr_col
    mask = in_row.astype(jnp.float32)
    rmat = jnp.logical_and(in_row, in_col).astype(jnp.float32)
    mask8 = mask.astype(jnp.float8_e4m3fn)
    m_ref[...] = mask8
    rmat8 = rmat.astype(jnp.float8_e4m3fn)
    r_ref[...] = rmat8
    ones = jnp.full((mask8.shape[1], 128), 1.0, jnp.float8_e4m3fn)
    s_part = jax.lax.dot_general(mask8, ones, (((1,), (0,)), ((), ())),
                                 preferred_element_type=jnp.float32)
    r_part = jax.lax.dot_general(rmat8, ones, (((1,), (0,)), ((), ())),
                                 preferred_element_type=jnp.float32)

    @pl.when(j == 0)
    def _init():
        s_ref[...] = s_part
        rs_ref[...] = r_part

    @pl.when(j != 0)
    def _acc():
        s_ref[...] += s_part
        rs_ref[...] += r_part


def _run_k2(z, kth_r, kth_c, bm=1024, bn=2048):
    gi, gj = N // bm, N // bn
    return pl.pallas_call(
        _k2_kernel,
        grid=(gi, gj),
        in_specs=[
            pl.BlockSpec((bm, D), lambda i, j: (i, 0)),
            pl.BlockSpec((bn, D), lambda i, j: (j, 0)),
            pl.BlockSpec((1, bn), lambda i, j: (0, j)),
            pl.BlockSpec((bm, 1), lambda i, j: (i, 0)),
        ],
        out_specs=[
            pl.BlockSpec((bm, bn), lambda i, j: (i, j)),
            pl.BlockSpec((bm, bn), lambda i, j: (i, j)),
            pl.BlockSpec((bm, 128), lambda i, j: (i, 0)),
            pl.BlockSpec((bm, 128), lambda i, j: (i, 0)),
        ],
        out_shape=[
            jax.ShapeDtypeStruct((N, N), jnp.float8_e4m3fn),
            jax.ShapeDtypeStruct((N, N), jnp.float8_e4m3fn),
            jax.ShapeDtypeStruct((N, 128), jnp.float32),
            jax.ShapeDtypeStruct((N, 128), jnp.float32),
        ],
        compiler_params=pltpu.CompilerParams(
            dimension_semantics=("parallel", "arbitrary")),
    )(z, z, kth_r, kth_c)


def _k3_kernel(ri_ref, mj_ref, h_ref):
    h = jax.lax.dot_general(
        ri_ref[...], mj_ref[...], (((1,), (0,)), ((), ())),
        preferred_element_type=jnp.float32)
    h_ref[...] = h.astype(jnp.float8_e4m3fn)


def _run_k3(r, m, bm=1024, bn=2048):
    gi, gj = N // bm, N // bn
    return pl.pallas_call(
        _k3_kernel,
        grid=(gi, gj),
        in_specs=[
            pl.BlockSpec((bm, N), lambda i, j: (i, 0)),
            pl.BlockSpec((N, bn), lambda i, j: (0, j)),
        ],
        out_specs=pl.BlockSpec((bm, bn), lambda i, j: (i, j)),
        out_shape=jax.ShapeDtypeStruct((N, N), jnp.float8_e4m3fn),
        compiler_params=pltpu.CompilerParams(
            dimension_semantics=("parallel", "parallel")),
    )(r, m)


def _k4_kernel(mi_ref, hj_ref, w_ref):
    w = jax.lax.dot_general(
        mi_ref[...], hj_ref[...], (((1,), (1,)), ((), ())),
        preferred_element_type=jnp.float32)
    w_ref[...] = w.astype(jnp.bfloat16)


def _run_k4(m, h, bm=1024, bn=2048):
    gi, gj = N // bm, N // bn
    return pl.pallas_call(
        _k4_kernel,
        grid=(gi, gj),
        in_specs=[
            pl.BlockSpec((bm, N), lambda i, j: (i, 0)),
            pl.BlockSpec((bn, N), lambda i, j: (j, 0)),
        ],
        out_specs=pl.BlockSpec((bm, bn), lambda i, j: (i, j)),
        out_shape=jax.ShapeDtypeStruct((N, N), jnp.bfloat16),
        compiler_params=pltpu.CompilerParams(
            dimension_semantics=("parallel", "parallel")),
    )(m, h)


def _k5_kernel(a_ref, b_ref, si_ref, ri_ref, sj_ref, rj_ref, out_ref):
    u_i = 1.0 / (si_ref[:, :1] * ri_ref[:, :1])
    u_j = 1.0 / (sj_ref[:, :1] * rj_ref[:, :1])
    a = a_ref[...].astype(jnp.float32)
    b = b_ref[...].astype(jnp.float32)
    out_ref[...] = 0.5 * (a * u_i + (b * u_j).T)


def _run_k5(w, s, rs, b=1024):
    g = N // b
    return pl.pallas_call(
        _k5_kernel,
        grid=(g, g),
        in_specs=[
            pl.BlockSpec((b, b), lambda i, j: (i, j)),
            pl.BlockSpec((b, b), lambda i, j: (j, i)),
            pl.BlockSpec((b, 128), lambda i, j: (i, 0)),
            pl.BlockSpec((b, 128), lambda i, j: (i, 0)),
            pl.BlockSpec((b, 128), lambda i, j: (j, 0)),
            pl.BlockSpec((b, 128), lambda i, j: (j, 0)),
        ],
        out_specs=pl.BlockSpec((b, b), lambda i, j: (i, j)),
        out_shape=jax.ShapeDtypeStruct((N, N), jnp.float32),
        compiler_params=pltpu.CompilerParams(
            dimension_semantics=("parallel", "parallel")),
    )(w, w, s, rs, s, rs)


@jax.jit
def kernel(z):
    kth = _run_k1(z)
    kth_flat = kth.reshape(N)
    kth_r = kth_flat.reshape(1, N)
    kth_c = kth_flat.reshape(N, 1)
    m, r, s, rs = _run_k2(z, kth_r, kth_c)
    h = _run_k3(r, m)
    w = _run_k4(m, h)
    return _run_k5(w, s, rs)
